# baseline (speedup 1.0000x reference)
import functools

import jax
import jax.numpy as jnp
from jax import lax
from jax.experimental import pallas as pl
from jax.experimental.pallas import tpu as pltpu
from jax.experimental.pallas import tpu_sc as plsc

_C = 32


def kernel(x, pe_weight, pos):
    B, L, D = x.shape
    V, P = pe_weight.shape
    W = D + P
    R = B * L
    info = plsc.get_sparse_core_info()
    nw = info.num_cores * info.num_subcores
    rows_w = R // nw
    nchunk = rows_w // _C
    x2 = x.reshape(R, D)
    mesh = plsc.VectorSubcoreMesh(core_axis_name="c", subcore_axis_name="s")

    @functools.partial(
        pl.kernel,
        mesh=mesh,
        out_type=jax.ShapeDtypeStruct((R, W), x.dtype),
        scratch_types=[
            pltpu.VMEM((rows_w,), jnp.int32),
            pltpu.VMEM((2, _C, D), jnp.float32),
            pltpu.VMEM((2, _C, P), jnp.float32),
            pltpu.SemaphoreType.DMA,
            pltpu.SemaphoreType.DMA,
            pltpu.SemaphoreType.DMA,
            pltpu.SemaphoreType.DMA,
            pltpu.SemaphoreType.DMA,
            pltpu.SemaphoreType.DMA,
            pltpu.SemaphoreType.DMA,
            pltpu.SemaphoreType.DMA,
        ],
    )
    def sck(x_hbm, pe_hbm, pos_hbm, out_hbm, idx_v, xbuf, pbuf,
            sx0, sx1, sp0, sp1, sox0, sox1, sop0, sop1):
        semx = (sx0, sx1)
        semp = (sp0, sp1)
        semox = (sox0, sox1)
        semop = (sop0, sop1)
        wid = lax.axis_index("s") * info.num_cores + lax.axis_index("c")
        rbase = wid * rows_w
        lbase = lax.rem(rbase, L)
        pltpu.sync_copy(pos_hbm.at[pl.ds(lbase, rows_w)], idx_v)

        def start_in(k):
            slot = k % 2
            r0 = rbase + k * _C
            cx = pltpu.async_copy(x_hbm.at[pl.ds(r0, _C)], xbuf.at[slot],
                                  semx[slot])
            cp = pltpu.async_copy(pe_hbm.at[idx_v.at[pl.ds(k * _C, _C)]],
                                  pbuf.at[slot], semp[slot])
            return cx, cp

        def start_out(k):
            slot = k % 2
            r0 = rbase + k * _C
            ox = pltpu.async_copy(xbuf.at[slot],
                                  out_hbm.at[pl.ds(r0, _C), pl.ds(0, D)],
                                  semox[slot])
            op = pltpu.async_copy(pbuf.at[slot],
                                  out_hbm.at[pl.ds(r0, _C), pl.ds(D, P)],
                                  semop[slot])
            return ox, op

        inflight_in = {0: start_in(0)}
        inflight_out = {}
        for k in range(nchunk):
            for c in inflight_in.pop(k):
                c.wait()
            inflight_out[k] = start_out(k)
            if k + 1 < nchunk:
                if k >= 1:
                    for c in inflight_out.pop(k - 1):
                        c.wait()
                inflight_in[k + 1] = start_in(k + 1)
        for k in list(inflight_out):
            for c in inflight_out.pop(k):
                c.wait()

    out2 = sck(x2, pe_weight, pos)
    return out2.reshape(B, L, W)


# speedup vs baseline: 1.1269x; 1.1269x over previous
import functools

import jax
import jax.numpy as jnp
from jax import lax
from jax.experimental import pallas as pl
from jax.experimental.pallas import tpu as pltpu
from jax.experimental.pallas import tpu_sc as plsc

_BLK = 2048


def _concat_body(x_ref, pe_ref, out_ref):
    d = x_ref.shape[2]
    out_ref[0, :, :d] = x_ref[0]
    out_ref[0, :, d:] = pe_ref[...]


def _tc_concat(x, x_pos):
    B, L, D = x.shape
    P = x_pos.shape[1]
    grid = (L // _BLK, B)
    return pl.pallas_call(
        _concat_body,
        grid=grid,
        in_specs=[
            pl.BlockSpec((1, _BLK, D), lambda i, b: (b, i, 0)),
            pl.BlockSpec((_BLK, P), lambda i, b: (i, 0)),
        ],
        out_specs=pl.BlockSpec((1, _BLK, D + P), lambda i, b: (b, i, 0)),
        out_shape=jax.ShapeDtypeStruct((B, L, D + P), x.dtype),
        compiler_params=pltpu.CompilerParams(
            dimension_semantics=("parallel", "parallel"),
        ),
    )(x, x_pos)


def _sc_gather(pe_weight, pos):
    V, P = pe_weight.shape
    L = pos.shape[0]
    info = plsc.get_sparse_core_info()
    nw = info.num_cores * info.num_subcores
    rows_per_w = L // nw
    mesh = plsc.VectorSubcoreMesh(core_axis_name="c", subcore_axis_name="s")

    @functools.partial(
        pl.kernel,
        mesh=mesh,
        out_type=jax.ShapeDtypeStruct((L, P), pe_weight.dtype),
        scratch_types=[
            pltpu.VMEM((rows_per_w,), jnp.int32),
            pltpu.VMEM((rows_per_w, P), pe_weight.dtype),
            pltpu.SemaphoreType.DMA,
        ],
    )
    def gather_k(pe_hbm, pos_hbm, out_hbm, idx_v, rows_v, sem):
        wid = lax.axis_index("s") * info.num_cores + lax.axis_index("c")
        base = wid * rows_per_w
        pltpu.sync_copy(pos_hbm.at[pl.ds(base, rows_per_w)], idx_v)
        pltpu.async_copy(pe_hbm.at[idx_v], rows_v, sem).wait()
        pltpu.sync_copy(rows_v, out_hbm.at[pl.ds(base, rows_per_w)])

    return gather_k(pe_weight, pos)


def kernel(x, pe_weight, pos):
    x_pos = _sc_gather(pe_weight, pos)
    return _tc_concat(x, x_pos)
